# Initial kernel scaffold; baseline (speedup 1.0000x reference)
import jax
import jax.numpy as jnp
from jax.experimental import pallas as pl


def kernel(x, W, b):
    raise NotImplementedError("write your pallas kernel here")


# speedup vs baseline: 18.0319x; 18.0319x over previous
import functools

import jax
import jax.numpy as jnp
from jax.experimental import pallas as pl
from jax.experimental.pallas import tpu as pltpu

_TOPK = 100
_IMG = 640.0
_N = 6400
_LANES = 128
_ROWS = _N // _LANES
_RPAD = 64
_BIGIDX = 1 << 30


def _beats(ak, ai, bk, bi):
    return (ak > bk) | ((ak == bk) & (ai < bi))


def _stage(arrs, d, take_max):
    lane = jax.lax.broadcasted_iota(jnp.int32, arrs[0].shape, 1)
    hi = (lane & d) != 0
    rolled = [jnp.where(hi, jnp.roll(a, d, axis=1), jnp.roll(a, -d, axis=1))
              for a in arrs]
    win = _beats(arrs[0], arrs[1], rolled[0], rolled[1])
    sel = win == take_max
    return [jnp.where(sel, a, p) for a, p in zip(arrs, rolled)]


def _sort128_desc(arrs):
    lane = jax.lax.broadcasted_iota(jnp.int32, arrs[0].shape, 1)
    for k in range(1, 8):
        desc = (lane >> k) & 1 == 0
        for j in reversed(range(k)):
            d = 1 << j
            low = (lane & d) == 0
            arrs = _stage(arrs, d, low == desc)
    return arrs


def _rev_lanes(a):
    lane = jax.lax.broadcasted_iota(jnp.int32, a.shape, 1)
    for j in range(7):
        d = 1 << j
        hi = (lane & d) != 0
        a = jnp.where(hi, jnp.roll(a, d, axis=1), jnp.roll(a, -d, axis=1))
    return a


def _merge_cleanup(arrs):
    lane = jax.lax.broadcasted_iota(jnp.int32, arrs[0].shape, 1)
    for j in reversed(range(7)):
        d = 1 << j
        arrs = _stage(arrs, d, (lane & d) == 0)
    return arrs


_CHUNKS = 10
_CH_ANC = _N // _CHUNKS
_CH_ROWS = 80 // _CHUNKS


def _yolo_kernel(x_ref, w_ref, b_ref, out_ref, preds_ref):
    j = pl.program_id(1)
    xb = x_ref[0]
    scaled = xb / 255.0
    sb = scaled.astype(jnp.bfloat16)
    patches = sb.reshape(3, _CH_ROWS, 8, 80, 8).transpose(1, 3, 0, 2, 4)
    patches = patches.reshape(_CH_ANC, 192)
    acc = jnp.dot(patches, w_ref[...], preferred_element_type=jnp.float32)
    preds = acc + b_ref[0][None, :]
    preds_ref[:, pl.ds(j * _CH_ANC, _CH_ANC)] = preds.T

    @pl.when(j == _CHUNKS - 1)
    def _topk():
        _topk_decode(preds_ref, out_ref)


def _topk_decode(preds_ref, out_ref):
    chans = [preds_ref[c, :].reshape(_ROWS, _LANES) for c in range(6)]
    idx = jax.lax.broadcasted_iota(jnp.int32, (_ROWS, _LANES), 0) * _LANES \
        + jax.lax.broadcasted_iota(jnp.int32, (_ROWS, _LANES), 1)
    key = chans[4]
    pad_rows = _RPAD - _ROWS
    neg = jnp.full((pad_rows, _LANES), -jnp.inf, jnp.float32)
    key = jnp.concatenate([key, neg], axis=0)
    idx = jnp.concatenate(
        [idx, jnp.full((pad_rows, _LANES), _BIGIDX, jnp.int32)], axis=0)
    zpad = jnp.zeros((pad_rows, _LANES), jnp.float32)
    chans = [jnp.concatenate([c, zpad], axis=0) for c in chans]

    arrs = [key, idx] + chans
    arrs = _sort128_desc(arrs)
    rows = _RPAD
    while rows > 1:
        half = rows // 2
        a = [v.reshape(half, 2, _LANES)[:, 0, :] for v in arrs]
        b = [_rev_lanes(v.reshape(half, 2, _LANES)[:, 1, :]) for v in arrs]
        win = _beats(a[0], a[1], b[0], b[1])
        arrs = [jnp.where(win, av, bv) for av, bv in zip(a, b)]
        arrs = _merge_cleanup(arrs)
        rows = half

    v0, v1, v2, v3, conf, cls = [v[0] for v in arrs[2:]]
    x1 = v0 / _IMG
    y1 = v1 / _IMG
    x2 = v2 / _IMG
    y2 = v3 / _IMG
    w_norm = x2 - x1
    h_norm = y2 - y1
    cx = (x1 + x2) / 2.0
    cy = (y1 + y2) / 2.0
    res = jnp.stack([cx, cy, w_norm, h_norm, conf, cls], axis=0)
    out_ref[0] = res.T[:_TOPK, :]


@jax.jit
def kernel(x, W, b):
    B = x.shape[0]
    wm = W[:, ::-1, :, :].reshape(6, 192).T.astype(jnp.bfloat16)
    grid = (B, _CHUNKS)
    return pl.pallas_call(
        _yolo_kernel,
        grid=grid,
        in_specs=[
            pl.BlockSpec((1, 3, 8 * _CH_ROWS, 640), lambda i, j: (i, 0, j, 0)),
            pl.BlockSpec((192, 6), lambda i, j: (0, 0)),
            pl.BlockSpec((1, 6), lambda i, j: (0, 0)),
        ],
        out_specs=pl.BlockSpec((1, _TOPK, 6), lambda i, j: (i, 0, 0)),
        out_shape=jax.ShapeDtypeStruct((B, _TOPK, 6), jnp.float32),
        scratch_shapes=[pltpu.VMEM((6, _N), jnp.float32)],
    )(x, wm, b.reshape(1, 6))
